# baseline (speedup 1.0000x reference)
import functools

import jax
import jax.numpy as jnp
from jax import lax
from jax.experimental import pallas as pl
from jax.experimental.pallas import tpu as pltpu
from jax.experimental.pallas import tpu_sc as plsc

EMBED_DIM = 64
CHUNK = 128


@functools.cache
def _make_gather(n_rows: int, vocab: int, d: int):
    info = plsc.get_sparse_core_info()
    nc, ns = info.num_cores, info.num_subcores
    nw = nc * ns
    rows_per_w = n_rows // nw
    n_chunks = rows_per_w // CHUNK
    assert rows_per_w % CHUNK == 0 and n_rows % nw == 0

    mesh = plsc.VectorSubcoreMesh(core_axis_name="c", subcore_axis_name="s")

    @functools.partial(
        pl.kernel,
        out_type=jax.ShapeDtypeStruct((n_rows, d), jnp.float32),
        mesh=mesh,
        compiler_params=pltpu.CompilerParams(use_tc_tiling_on_sc=False),
        scratch_types=[
            pltpu.VMEM((n_chunks, CHUNK), jnp.int32),
            pltpu.VMEM((2, CHUNK, d), jnp.float32),
            pltpu.SemaphoreType.DMA,
            pltpu.SemaphoreType.DMA,
        ],
    )
    def gather_kernel(idx_hbm, table_hbm, out_hbm, idx_v, rows_v, sem0, sem1):
        wid = lax.axis_index("s") * nc + lax.axis_index("c")
        base = wid * rows_per_w

        pltpu.sync_copy(idx_hbm.at[pl.ds(wid * n_chunks, n_chunks)], idx_v)

        sems = (sem0, sem1)

        def start(j, b):
            pltpu.make_async_copy(
                table_hbm.at[idx_v.at[j]], rows_v.at[b], sems[b]
            ).start()

        def finish(j, b):
            pltpu.make_async_copy(
                table_hbm.at[idx_v.at[j]], rows_v.at[b], sems[b]
            ).wait()
            pltpu.sync_copy(rows_v.at[b], out_hbm.at[pl.ds(base + j * CHUNK, CHUNK)])

        start(0, 0)

        @pl.loop(0, n_chunks - 2, step=2)
        def _(j):
            start(j + 1, 1)
            finish(j, 0)
            start(j + 2, 0)
            finish(j + 1, 1)

        start(n_chunks - 1, 1)
        finish(n_chunks - 2, 0)
        finish(n_chunks - 1, 1)

    return gather_kernel


def kernel(inputs, table):
    b, h, _ = inputs.shape
    vocab, d = table.shape
    n_rows = b * h
    idx = inputs.reshape(n_rows // CHUNK, CHUNK).astype(jnp.int32)
    out = _make_gather(n_rows, vocab, d)(idx, table)
    return out.reshape(b, h, d)


# speedup vs baseline: 1.0193x; 1.0193x over previous
import functools

import jax
import jax.numpy as jnp
from jax import lax
from jax.experimental import pallas as pl
from jax.experimental.pallas import tpu as pltpu
from jax.experimental.pallas import tpu_sc as plsc

EMBED_DIM = 64
CHUNK = 512


@functools.cache
def _make_gather(n_rows: int, vocab: int, d: int):
    info = plsc.get_sparse_core_info()
    nc, ns = info.num_cores, info.num_subcores
    nw = nc * ns
    rows_per_w = n_rows // nw
    n_chunks = rows_per_w // CHUNK
    assert rows_per_w % CHUNK == 0 and n_rows % nw == 0

    mesh = plsc.VectorSubcoreMesh(core_axis_name="c", subcore_axis_name="s")

    @functools.partial(
        pl.kernel,
        out_type=jax.ShapeDtypeStruct((n_rows, d), jnp.float32),
        mesh=mesh,
        compiler_params=pltpu.CompilerParams(use_tc_tiling_on_sc=False),
        scratch_types=[
            pltpu.VMEM((n_chunks, CHUNK), jnp.int32),
            pltpu.VMEM((2, CHUNK, d), jnp.float32),
            pltpu.SemaphoreType.DMA,
            pltpu.SemaphoreType.DMA,
        ],
    )
    def gather_kernel(idx_hbm, table_hbm, out_hbm, idx_v, rows_v, sem0, sem1):
        wid = lax.axis_index("s") * nc + lax.axis_index("c")
        base = wid * rows_per_w

        pltpu.sync_copy(idx_hbm.at[pl.ds(wid * n_chunks, n_chunks)], idx_v)

        sems = (sem0, sem1)

        def start(j, b):
            pltpu.make_async_copy(
                table_hbm.at[idx_v.at[j]], rows_v.at[b], sems[b]
            ).start()

        def finish(j, b):
            pltpu.make_async_copy(
                table_hbm.at[idx_v.at[j]], rows_v.at[b], sems[b]
            ).wait()
            pltpu.sync_copy(rows_v.at[b], out_hbm.at[pl.ds(base + j * CHUNK, CHUNK)])

        start(0, 0)

        @pl.loop(0, n_chunks - 2, step=2)
        def _(j):
            start(j + 1, 1)
            finish(j, 0)
            start(j + 2, 0)
            finish(j + 1, 1)

        start(n_chunks - 1, 1)
        finish(n_chunks - 2, 0)
        finish(n_chunks - 1, 1)

    return gather_kernel


def kernel(inputs, table):
    b, h, _ = inputs.shape
    vocab, d = table.shape
    n_rows = b * h
    idx = inputs.reshape(n_rows // CHUNK, CHUNK).astype(jnp.int32)
    out = _make_gather(n_rows, vocab, d)(idx, table)
    return out.reshape(b, h, d)


# speedup vs baseline: 1.0193x; 1.0000x over previous
import functools

import jax
import jax.numpy as jnp
from jax import lax
from jax.experimental import pallas as pl
from jax.experimental.pallas import tpu as pltpu
from jax.experimental.pallas import tpu_sc as plsc

EMBED_DIM = 64
CHUNK = 256
NBUF = 5
LOOKAHEAD = 2


@functools.cache
def _make_gather(n_rows: int, vocab: int, d: int):
    info = plsc.get_sparse_core_info()
    nc, ns = info.num_cores, info.num_subcores
    nw = nc * ns
    rows_per_w = n_rows // nw
    n_chunks = rows_per_w // CHUNK
    assert rows_per_w % CHUNK == 0 and n_rows % nw == 0

    mesh = plsc.VectorSubcoreMesh(core_axis_name="c", subcore_axis_name="s")

    assert n_chunks % NBUF == 0 and n_chunks >= 2 * NBUF

    @functools.partial(
        pl.kernel,
        out_type=jax.ShapeDtypeStruct((n_rows, d), jnp.float32),
        mesh=mesh,
        compiler_params=pltpu.CompilerParams(use_tc_tiling_on_sc=False),
        scratch_types=[
            pltpu.VMEM((n_chunks, CHUNK), jnp.int32),
            pltpu.VMEM((NBUF, CHUNK, d), jnp.float32),
            [pltpu.SemaphoreType.DMA] * NBUF,
            [pltpu.SemaphoreType.DMA] * NBUF,
        ],
    )
    def gather_kernel(idx_hbm, table_hbm, out_hbm, idx_v, rows_v, gsems, wsems):
        wid = lax.axis_index("s") * nc + lax.axis_index("c")
        base = wid * rows_per_w

        pltpu.sync_copy(idx_hbm.at[pl.ds(wid * n_chunks, n_chunks)], idx_v)

        def gather(j, b):
            return pltpu.make_async_copy(
                table_hbm.at[idx_v.at[j]], rows_v.at[b], gsems[b]
            )

        def write(j, b):
            return pltpu.make_async_copy(
                rows_v.at[b], out_hbm.at[pl.ds(base + j * CHUNK, CHUNK)], wsems[b]
            )

        def consume(j):
            b = j % NBUF
            gather(j, b).wait()
            write(j, b).start()
            jn = j + LOOKAHEAD
            if jn < n_chunks:
                bn = jn % NBUF
                if jn - NBUF >= 0:
                    write(jn - NBUF, bn).wait()
                gather(jn, bn).start()

        for j in range(LOOKAHEAD):
            gather(j, j % NBUF).start()

        head = NBUF
        tail = ((n_chunks - LOOKAHEAD) // NBUF) * NBUF
        for j in range(head):
            consume(j)

        @pl.loop(head, tail, step=NBUF)
        def _(j0):
            for boff in range(NBUF):
                j = j0 + boff
                b = (head + boff) % NBUF
                gather(j, b).wait()
                write(j, b).start()
                jn = j + LOOKAHEAD
                bn = (b + LOOKAHEAD) % NBUF
                write(jn - NBUF, bn).wait()
                gather(jn, bn).start()

        for j in range(tail, n_chunks):
            consume(j)

        for j in range(n_chunks - NBUF, n_chunks):
            write(j, j % NBUF).wait()

    return gather_kernel


def kernel(inputs, table):
    b, h, _ = inputs.shape
    vocab, d = table.shape
    n_rows = b * h
    idx = inputs.reshape(n_rows // CHUNK, CHUNK).astype(jnp.int32)
    out = _make_gather(n_rows, vocab, d)(idx, table)
    return out.reshape(b, h, d)


# speedup vs baseline: 1.0217x; 1.0023x over previous
import functools

import jax
import jax.numpy as jnp
from jax import lax
from jax.experimental import pallas as pl
from jax.experimental.pallas import tpu as pltpu
from jax.experimental.pallas import tpu_sc as plsc

NBUF = 4
LOOKAHEAD = 2


@functools.cache
def _make_gather(b: int, h: int, vocab: int, d: int):
    info = plsc.get_sparse_core_info()
    nc, ns = info.num_cores, info.num_subcores
    nw = nc * ns
    rows_per_w = b // nw
    n_chunks = rows_per_w
    assert b % nw == 0 and n_chunks % NBUF == 0 and n_chunks >= 2 * NBUF

    mesh = plsc.VectorSubcoreMesh(core_axis_name="c", subcore_axis_name="s")

    @functools.partial(
        pl.kernel,
        out_type=jax.ShapeDtypeStruct((b, h, d), jnp.float32),
        mesh=mesh,
        compiler_params=pltpu.CompilerParams(use_tc_tiling_on_sc=False),
        scratch_types=[
            pltpu.VMEM((rows_per_w, h), jnp.int32),
            pltpu.VMEM((NBUF, h, d), jnp.float32),
            [pltpu.SemaphoreType.DMA] * NBUF,
            [pltpu.SemaphoreType.DMA] * NBUF,
        ],
    )
    def gather_kernel(idx_hbm, table_hbm, out_hbm, idx_v, rows_v, gsems, wsems):
        wid = lax.axis_index("s") * nc + lax.axis_index("c")
        base = wid * rows_per_w

        pltpu.sync_copy(idx_hbm.at[pl.ds(base, rows_per_w)], idx_v)

        def gather(j, bf):
            return pltpu.make_async_copy(
                table_hbm.at[idx_v.at[j]], rows_v.at[bf], gsems[bf]
            )

        def write(j, bf):
            return pltpu.make_async_copy(
                rows_v.at[bf], out_hbm.at[base + j], wsems[bf]
            )

        def consume(j):
            bf = j % NBUF
            gather(j, bf).wait()
            write(j, bf).start()
            jn = j + LOOKAHEAD
            if jn < n_chunks:
                bn = jn % NBUF
                if jn - NBUF >= 0:
                    write(jn - NBUF, bn).wait()
                gather(jn, bn).start()

        for j in range(LOOKAHEAD):
            gather(j, j % NBUF).start()

        head = NBUF
        tail = ((n_chunks - LOOKAHEAD) // NBUF) * NBUF
        for j in range(head):
            consume(j)

        @pl.loop(head, tail, step=NBUF)
        def _(j0):
            for boff in range(NBUF):
                j = j0 + boff
                bf = boff
                gather(j, bf).wait()
                write(j, bf).start()
                jn = j + LOOKAHEAD
                bn = (bf + LOOKAHEAD) % NBUF
                write(jn - NBUF, bn).wait()
                gather(jn, bn).start()

        for j in range(tail, n_chunks):
            consume(j)

        for j in range(n_chunks - NBUF, n_chunks):
            write(j, j % NBUF).wait()

    return gather_kernel


def kernel(inputs, table):
    b, h, _ = inputs.shape
    vocab, d = table.shape
    idx = inputs[:, :, 0].astype(jnp.int32)
    return _make_gather(b, h, vocab, d)(idx, table)
